# baseline (speedup 1.0000x reference)
import jax
import jax.numpy as jnp
from jax import lax
from jax.experimental import pallas as pl
from jax.experimental.pallas import tpu as pltpu
from jax.experimental.pallas import tpu_sc as plsc

N = 100000
D = 128
NUM_SEGMENTS = 1024
NW = 32
SEG_PER_W = NUM_SEGMENTS // NW
CHUNK = 256
GROUPS = CHUNK // 16
L = 16
DC = D // L
ROW_W = 2 * D
_RUN_COMPUTE = True
_ABLATE_SIGMOID = False
_ABLATE_MAX = True

_GDN = lax.GatherDimensionNumbers(
    offset_dims=(), collapsed_slice_dims=(0,), start_index_map=(0,))


def _perm(v, p):
    return lax.gather(v, p.reshape(L, 1), _GDN, (1,),
                      mode=lax.GatherScatterMode.PROMISE_IN_BOUNDS)


def _bcast_lane(v, j):
    return _perm(v, jnp.full((L,), j, dtype=jnp.int32))


def _allsum(v):
    lanes = jnp.arange(L, dtype=jnp.int32)
    for s in (8, 4, 2, 1):
        v = v + _perm(v, jnp.bitwise_xor(lanes, s))
    return v


def _body(feats_hbm, seg_hbm, params_hbm, bounds_hbm, out_hbm,
          fbufA, fbufB, sbufA, sbufB, acc, wbuf, bbuf, semA, semB):
    wid = lax.axis_index("s") * 2 + lax.axis_index("c")

    pltpu.sync_copy(params_hbm, wbuf)
    pltpu.sync_copy(bounds_hbm, bbuf)

    bv = bbuf[pl.ds(wid, L)]
    row_start = bv[0]
    row_end = bv[1]

    wv = [wbuf[pl.ds(c * L, L)] for c in range(DC)]
    b_v = _bcast_lane(wbuf[pl.ds(D, L)], 0)

    iota = jnp.arange(L, dtype=jnp.int32)
    segbase = wid * SEG_PER_W
    segbase_v = jnp.full((L,), segbase, jnp.int32)
    col_sum = [iota + c * L for c in range(DC)]
    col_max = [iota + (D + c * L) for c in range(DC)]
    zeros_v = jnp.zeros((L,), jnp.float32)
    neginf_v = jnp.full((L,), -jnp.inf, jnp.float32)

    def init_row(i, carry):
        for c in range(DC):
            acc[pl.ds(i * ROW_W + c * L, L)] = zeros_v
            acc[pl.ds(i * ROW_W + D + c * L, L)] = neginf_v
        return carry
    lax.fori_loop(0, SEG_PER_W + 1, init_row, 0)

    base0 = pl.multiple_of(jnp.bitwise_and(row_start, -8), 8)
    n_chunks = jnp.maximum(
        (row_end - base0 + CHUNK - 1) // CHUNK, 0)

    def chunk_base(k):
        return pl.multiple_of(jnp.minimum(base0 + k * CHUNK, N - CHUNK), 8)

    def start(k, fb, sb, sem):
        b = chunk_base(k)
        pltpu.async_copy(feats_hbm.at[pl.ds(b, CHUNK)], fb, sem)
        pltpu.async_copy(seg_hbm.at[pl.ds(b, CHUNK)], sb, sem)

    def wait(fb, sb, sem):
        pltpu.make_async_copy(feats_hbm.at[pl.ds(0, CHUNK)], fb, sem).wait()
        pltpu.make_async_copy(seg_hbm.at[pl.ds(0, CHUNK)], sb, sem).wait()

    def process(k, fb, sb):
        bk = base0 + k * CHUNK
        base = chunk_base(k)
        lo = jnp.maximum(bk, row_start)
        hi = jnp.minimum(bk + CHUNK, row_end)

        def group(g, carry):
            seg_vec = sb[pl.ds(g * L, L)]
            g_lo = base + g * L
            uniform = jnp.logical_and(
                jnp.all(seg_vec == _bcast_lane(seg_vec, 0)),
                jnp.logical_and(g_lo >= lo, g_lo + L <= hi))

            def fast(_):
                s_off = (seg_vec[0] - segbase) * ROW_W
                gsum = [zeros_v] * DC
                gmax = [neginf_v] * DC
                for j in range(L):
                    row = g * L + j
                    x = [fb[row, pl.ds(c * L, L)] for c in range(DC)]
                    dot = x[0] * wv[0]
                    for c in range(1, DC):
                        dot = dot + x[c] * wv[c]
                    z_v = _allsum(dot) + b_v
                    if _ABLATE_SIGMOID:
                        gate = z_v * 0.25
                    else:
                        gate = 1.0 / (1.0 + jnp.exp(-z_v))
                    for c in range(DC):
                        gsum[c] = gsum[c] + x[c] * gate
                        if not _ABLATE_MAX:
                            gmax[c] = jnp.maximum(gmax[c], x[c])
                for c in range(DC):
                    ds_s = pl.ds(s_off + c * L, L)
                    acc[ds_s] = acc[ds_s] + gsum[c]
                    ds_m = pl.ds(s_off + D + c * L, L)
                    acc[ds_m] = jnp.maximum(acc[ds_m], gmax[c])
                return 0

            def slow(_):
                for j in range(L):
                    rg = g_lo + j
                    m = jnp.logical_and(rg >= lo, rg < hi)
                    mi_v = jnp.full((L,), m.astype(jnp.int32))
                    s_l = _bcast_lane(seg_vec, j) - segbase_v
                    s_cl = jnp.clip(s_l, 0, SEG_PER_W - 1)
                    idx_base = (SEG_PER_W + mi_v * (s_cl - SEG_PER_W)) * ROW_W

                    row = g * L + j
                    x = [fb[row, pl.ds(c * L, L)] for c in range(DC)]
                    dot = x[0] * wv[0]
                    for c in range(1, DC):
                        dot = dot + x[c] * wv[c]
                    z_v = _allsum(dot) + b_v
                    gate = 1.0 / (1.0 + jnp.exp(-z_v))

                    for c in range(DC):
                        plsc.addupdate_scatter(acc, [idx_base + col_sum[c]],
                                               x[c] * gate)
                    for c in range(DC):
                        idx = idx_base + col_max[c]
                        old = plsc.load_gather(acc, [idx])
                        plsc.store_scatter(acc, [idx],
                                           jnp.maximum(old, x[c]))
                return 0

            lax.cond(uniform, fast, slow, 0)
            return carry

        lax.fori_loop(0, GROUPS, group, 0)

    @pl.when(n_chunks > 0)
    def _prologue():
        start(0, fbufA, sbufA, semA)

    def pair(kk, carry):
        k0 = 2 * kk

        @pl.when(k0 + 1 < n_chunks)
        def _s1():
            start(k0 + 1, fbufB, sbufB, semB)
        wait(fbufA, sbufA, semA)
        if _RUN_COMPUTE:
            process(k0, fbufA, sbufA)

        @pl.when(k0 + 2 < n_chunks)
        def _s2():
            start(k0 + 2, fbufA, sbufA, semA)

        @pl.when(k0 + 1 < n_chunks)
        def _p1():
            wait(fbufB, sbufB, semB)
            if _RUN_COMPUTE:
                process(k0 + 1, fbufB, sbufB)
        return carry

    lax.fori_loop(0, (n_chunks + 1) // 2, pair, 0)

    pltpu.sync_copy(acc.at[pl.ds(0, SEG_PER_W * ROW_W)],
                    out_hbm.at[pl.ds(wid * SEG_PER_W * ROW_W,
                                     SEG_PER_W * ROW_W)])


@jax.jit
def kernel(feats, segment_ids, W, b):
    params = jnp.concatenate(
        [W.reshape(D), b.astype(jnp.float32),
         jnp.zeros((2 * L - 1,), jnp.float32)])
    seg_bounds = jnp.searchsorted(
        segment_ids,
        jnp.arange(0, NUM_SEGMENTS + 1, SEG_PER_W, dtype=jnp.int32),
    ).astype(jnp.int32)
    bounds = jnp.concatenate(
        [seg_bounds, jnp.full((15,), N, jnp.int32)])

    mesh = plsc.VectorSubcoreMesh(core_axis_name="c", subcore_axis_name="s",
                                  num_cores=2, num_subcores=16)
    run = pl.kernel(
        _body,
        out_type=jax.ShapeDtypeStruct((NUM_SEGMENTS * ROW_W,), jnp.float32),
        mesh=mesh,
        scratch_types=[
            pltpu.VMEM((CHUNK, D), jnp.float32),
            pltpu.VMEM((CHUNK, D), jnp.float32),
            pltpu.VMEM((CHUNK,), jnp.int32),
            pltpu.VMEM((CHUNK,), jnp.int32),
            pltpu.VMEM(((SEG_PER_W + 1) * ROW_W,), jnp.float32),
            pltpu.VMEM((D + 2 * L,), jnp.float32),
            pltpu.VMEM((3 * L,), jnp.int32),
            pltpu.SemaphoreType.DMA,
            pltpu.SemaphoreType.DMA,
        ],
        compiler_params=pltpu.CompilerParams(needs_layout_passes=False),
    )
    return run(feats, segment_ids, params, bounds).reshape(
        NUM_SEGMENTS, ROW_W)


# speedup vs baseline: 1.1646x; 1.1646x over previous
import jax
import jax.numpy as jnp
from jax import lax
from jax.experimental import pallas as pl
from jax.experimental.pallas import tpu as pltpu
from jax.experimental.pallas import tpu_sc as plsc

N = 100000
D = 128
NUM_SEGMENTS = 1024
NW = 32
SEG_PER_W = NUM_SEGMENTS // NW
CHUNK = 256
GROUPS = CHUNK // 16
L = 16
DC = D // L
ROW_W = 2 * D
_RUN_COMPUTE = True
_ABLATE_SIGMOID = False
_ABLATE_MAX = True
_ABLATE_DOT = True

_GDN = lax.GatherDimensionNumbers(
    offset_dims=(), collapsed_slice_dims=(0,), start_index_map=(0,))


def _perm(v, p):
    return lax.gather(v, p.reshape(L, 1), _GDN, (1,),
                      mode=lax.GatherScatterMode.PROMISE_IN_BOUNDS)


def _bcast_lane(v, j):
    return _perm(v, jnp.full((L,), j, dtype=jnp.int32))


def _allsum(v):
    lanes = jnp.arange(L, dtype=jnp.int32)
    for s in (8, 4, 2, 1):
        v = v + _perm(v, jnp.bitwise_xor(lanes, s))
    return v


def _body(feats_hbm, seg_hbm, params_hbm, bounds_hbm, out_hbm,
          fbufA, fbufB, sbufA, sbufB, acc, wbuf, bbuf, semA, semB):
    wid = lax.axis_index("s") * 2 + lax.axis_index("c")

    pltpu.sync_copy(params_hbm, wbuf)
    pltpu.sync_copy(bounds_hbm, bbuf)

    bv = bbuf[pl.ds(wid, L)]
    row_start = bv[0]
    row_end = bv[1]

    wv = [wbuf[pl.ds(c * L, L)] for c in range(DC)]
    b_v = _bcast_lane(wbuf[pl.ds(D, L)], 0)

    iota = jnp.arange(L, dtype=jnp.int32)
    segbase = wid * SEG_PER_W
    segbase_v = jnp.full((L,), segbase, jnp.int32)
    col_sum = [iota + c * L for c in range(DC)]
    col_max = [iota + (D + c * L) for c in range(DC)]
    zeros_v = jnp.zeros((L,), jnp.float32)
    neginf_v = jnp.full((L,), -jnp.inf, jnp.float32)

    def init_row(i, carry):
        for c in range(DC):
            acc[pl.ds(i * ROW_W + c * L, L)] = zeros_v
            acc[pl.ds(i * ROW_W + D + c * L, L)] = neginf_v
        return carry
    lax.fori_loop(0, SEG_PER_W + 1, init_row, 0)

    base0 = pl.multiple_of(jnp.bitwise_and(row_start, -8), 8)
    n_chunks = jnp.maximum(
        (row_end - base0 + CHUNK - 1) // CHUNK, 0)

    def chunk_base(k):
        return pl.multiple_of(jnp.minimum(base0 + k * CHUNK, N - CHUNK), 8)

    def start(k, fb, sb, sem):
        b = chunk_base(k)
        pltpu.async_copy(feats_hbm.at[pl.ds(b, CHUNK)], fb, sem)
        pltpu.async_copy(seg_hbm.at[pl.ds(b, CHUNK)], sb, sem)

    def wait(fb, sb, sem):
        pltpu.make_async_copy(feats_hbm.at[pl.ds(0, CHUNK)], fb, sem).wait()
        pltpu.make_async_copy(seg_hbm.at[pl.ds(0, CHUNK)], sb, sem).wait()

    def process(k, fb, sb):
        bk = base0 + k * CHUNK
        base = chunk_base(k)
        lo = jnp.maximum(bk, row_start)
        hi = jnp.minimum(bk + CHUNK, row_end)

        def group(g, carry):
            seg_vec = sb[pl.ds(g * L, L)]
            g_lo = base + g * L
            uniform = jnp.logical_and(
                jnp.all(seg_vec == _bcast_lane(seg_vec, 0)),
                jnp.logical_and(g_lo >= lo, g_lo + L <= hi))

            def fast(_):
                s_off = (seg_vec[0] - segbase) * ROW_W
                gsum = [zeros_v] * DC
                gmax = [neginf_v] * DC
                for j in range(L):
                    row = g * L + j
                    x = [fb[row, pl.ds(c * L, L)] for c in range(DC)]
                    if _ABLATE_DOT:
                        gate = b_v
                    else:
                        dot = x[0] * wv[0]
                        for c in range(1, DC):
                            dot = dot + x[c] * wv[c]
                        z_v = _allsum(dot) + b_v
                        if _ABLATE_SIGMOID:
                            gate = z_v * 0.25
                        else:
                            gate = 1.0 / (1.0 + jnp.exp(-z_v))
                    for c in range(DC):
                        gsum[c] = gsum[c] + x[c] * gate
                        if not _ABLATE_MAX:
                            gmax[c] = jnp.maximum(gmax[c], x[c])
                for c in range(DC):
                    ds_s = pl.ds(s_off + c * L, L)
                    acc[ds_s] = acc[ds_s] + gsum[c]
                    ds_m = pl.ds(s_off + D + c * L, L)
                    acc[ds_m] = jnp.maximum(acc[ds_m], gmax[c])
                return 0

            def slow(_):
                for j in range(L):
                    rg = g_lo + j
                    m = jnp.logical_and(rg >= lo, rg < hi)
                    mi_v = jnp.full((L,), m.astype(jnp.int32))
                    s_l = _bcast_lane(seg_vec, j) - segbase_v
                    s_cl = jnp.clip(s_l, 0, SEG_PER_W - 1)
                    idx_base = (SEG_PER_W + mi_v * (s_cl - SEG_PER_W)) * ROW_W

                    row = g * L + j
                    x = [fb[row, pl.ds(c * L, L)] for c in range(DC)]
                    dot = x[0] * wv[0]
                    for c in range(1, DC):
                        dot = dot + x[c] * wv[c]
                    z_v = _allsum(dot) + b_v
                    gate = 1.0 / (1.0 + jnp.exp(-z_v))

                    for c in range(DC):
                        plsc.addupdate_scatter(acc, [idx_base + col_sum[c]],
                                               x[c] * gate)
                    for c in range(DC):
                        idx = idx_base + col_max[c]
                        old = plsc.load_gather(acc, [idx])
                        plsc.store_scatter(acc, [idx],
                                           jnp.maximum(old, x[c]))
                return 0

            lax.cond(uniform, fast, slow, 0)
            return carry

        lax.fori_loop(0, GROUPS, group, 0)

    @pl.when(n_chunks > 0)
    def _prologue():
        start(0, fbufA, sbufA, semA)

    def pair(kk, carry):
        k0 = 2 * kk

        @pl.when(k0 + 1 < n_chunks)
        def _s1():
            start(k0 + 1, fbufB, sbufB, semB)
        wait(fbufA, sbufA, semA)
        if _RUN_COMPUTE:
            process(k0, fbufA, sbufA)

        @pl.when(k0 + 2 < n_chunks)
        def _s2():
            start(k0 + 2, fbufA, sbufA, semA)

        @pl.when(k0 + 1 < n_chunks)
        def _p1():
            wait(fbufB, sbufB, semB)
            if _RUN_COMPUTE:
                process(k0 + 1, fbufB, sbufB)
        return carry

    lax.fori_loop(0, (n_chunks + 1) // 2, pair, 0)

    pltpu.sync_copy(acc.at[pl.ds(0, SEG_PER_W * ROW_W)],
                    out_hbm.at[pl.ds(wid * SEG_PER_W * ROW_W,
                                     SEG_PER_W * ROW_W)])


@jax.jit
def kernel(feats, segment_ids, W, b):
    params = jnp.concatenate(
        [W.reshape(D), b.astype(jnp.float32),
         jnp.zeros((2 * L - 1,), jnp.float32)])
    seg_bounds = jnp.searchsorted(
        segment_ids,
        jnp.arange(0, NUM_SEGMENTS + 1, SEG_PER_W, dtype=jnp.int32),
    ).astype(jnp.int32)
    bounds = jnp.concatenate(
        [seg_bounds, jnp.full((15,), N, jnp.int32)])

    mesh = plsc.VectorSubcoreMesh(core_axis_name="c", subcore_axis_name="s",
                                  num_cores=2, num_subcores=16)
    run = pl.kernel(
        _body,
        out_type=jax.ShapeDtypeStruct((NUM_SEGMENTS * ROW_W,), jnp.float32),
        mesh=mesh,
        scratch_types=[
            pltpu.VMEM((CHUNK, D), jnp.float32),
            pltpu.VMEM((CHUNK, D), jnp.float32),
            pltpu.VMEM((CHUNK,), jnp.int32),
            pltpu.VMEM((CHUNK,), jnp.int32),
            pltpu.VMEM(((SEG_PER_W + 1) * ROW_W,), jnp.float32),
            pltpu.VMEM((D + 2 * L,), jnp.float32),
            pltpu.VMEM((3 * L,), jnp.int32),
            pltpu.SemaphoreType.DMA,
            pltpu.SemaphoreType.DMA,
        ],
        compiler_params=pltpu.CompilerParams(needs_layout_passes=False),
    )
    return run(feats, segment_ids, params, bounds).reshape(
        NUM_SEGMENTS, ROW_W)


# speedup vs baseline: 1.4922x; 1.2813x over previous
import jax
import jax.numpy as jnp
from jax import lax
from jax.experimental import pallas as pl
from jax.experimental.pallas import tpu as pltpu
from jax.experimental.pallas import tpu_sc as plsc

N = 100000
D = 128
NUM_SEGMENTS = 1024
NW = 32
SEG_PER_W = NUM_SEGMENTS // NW
CHUNK = 256
GROUPS = CHUNK // 16
L = 16
DC = D // L
ROW_W = 2 * D

_GDN = lax.GatherDimensionNumbers(
    offset_dims=(), collapsed_slice_dims=(0,), start_index_map=(0,))


def _perm(v, p):
    return lax.gather(v, p.reshape(L, 1), _GDN, (1,),
                      mode=lax.GatherScatterMode.PROMISE_IN_BOUNDS)


def _bcast_lane(v, j):
    return _perm(v, jnp.full((L,), j, dtype=jnp.int32))


def _allsum(v):
    lanes = jnp.arange(L, dtype=jnp.int32)
    for s in (8, 4, 2, 1):
        v = v + _perm(v, jnp.bitwise_xor(lanes, s))
    return v


def _body(feats_hbm, seg_hbm, params_hbm, bounds_hbm, out_hbm,
          fbufA, fbufB, sbufA, sbufB, acc, wbuf, bbuf, semA, semB):
    wid = lax.axis_index("s") * 2 + lax.axis_index("c")

    pltpu.sync_copy(params_hbm, wbuf)
    pltpu.sync_copy(bounds_hbm, bbuf)

    bv = bbuf[pl.ds(wid, L)]
    row_start = bv[0]
    row_end = bv[1]

    wv = [wbuf[pl.ds(c * L, L)] for c in range(DC)]
    b_v = _bcast_lane(wbuf[pl.ds(D, L)], 0)

    segbase = wid * SEG_PER_W
    zeros_v = jnp.zeros((L,), jnp.float32)
    neginf_v = jnp.full((L,), -jnp.inf, jnp.float32)

    def init_row(i, carry):
        for c in range(DC):
            acc[pl.ds(i * ROW_W + c * L, L)] = zeros_v
            acc[pl.ds(i * ROW_W + D + c * L, L)] = neginf_v
        return carry
    lax.fori_loop(0, SEG_PER_W + 1, init_row, 0)

    base0 = pl.multiple_of(jnp.bitwise_and(row_start, -8), 8)
    n_chunks = jnp.maximum(
        (row_end - base0 + CHUNK - 1) // CHUNK, 0)

    def chunk_base(k):
        return pl.multiple_of(jnp.minimum(base0 + k * CHUNK, N - CHUNK), 8)

    def start(k, fb, sb, sem):
        b = chunk_base(k)
        pltpu.async_copy(feats_hbm.at[pl.ds(b, CHUNK)], fb, sem)
        pltpu.async_copy(seg_hbm.at[pl.ds(b, CHUNK)], sb, sem)

    def wait(fb, sb, sem):
        pltpu.make_async_copy(feats_hbm.at[pl.ds(0, CHUNK)], fb, sem).wait()
        pltpu.make_async_copy(seg_hbm.at[pl.ds(0, CHUNK)], sb, sem).wait()

    def process(k, fb, sb):
        bk = base0 + k * CHUNK
        base = chunk_base(k)
        lo = jnp.maximum(bk, row_start)
        hi = jnp.minimum(bk + CHUNK, row_end)

        def group(g, carry):
            seg_vec = sb[pl.ds(g * L, L)]
            g_lo = base + g * L
            uniform = jnp.logical_and(
                jnp.all(seg_vec == _bcast_lane(seg_vec, 0)),
                jnp.logical_and(g_lo >= lo, g_lo + L <= hi))

            def fast(_):
                s_off = jnp.clip(seg_vec[0] - segbase, 0, SEG_PER_W - 1) \
                    * ROW_W
                gsum = [zeros_v] * DC
                gmax = [neginf_v] * DC
                for j in range(L):
                    row = g * L + j
                    x = [fb[row, pl.ds(c * L, L)] for c in range(DC)]
                    dot = x[0] * wv[0]
                    for c in range(1, DC):
                        dot = dot + x[c] * wv[c]
                    z_v = _allsum(dot) + b_v
                    gate = 1.0 / (1.0 + jnp.exp(-z_v))
                    for c in range(DC):
                        gsum[c] = gsum[c] + x[c] * gate
                        gmax[c] = jnp.maximum(gmax[c], x[c])
                for c in range(DC):
                    ds_s = pl.ds(s_off + c * L, L)
                    acc[ds_s] = acc[ds_s] + gsum[c]
                    ds_m = pl.ds(s_off + D + c * L, L)
                    acc[ds_m] = jnp.maximum(acc[ds_m], gmax[c])
                return 0

            def slow(_):
                for j in range(L):
                    rg = g_lo + j
                    m = jnp.logical_and(rg >= lo, rg < hi)
                    s_cl = jnp.clip(seg_vec[j] - segbase, 0, SEG_PER_W - 1)
                    off = jnp.where(m, s_cl, SEG_PER_W) * ROW_W

                    row = g * L + j
                    x = [fb[row, pl.ds(c * L, L)] for c in range(DC)]
                    dot = x[0] * wv[0]
                    for c in range(1, DC):
                        dot = dot + x[c] * wv[c]
                    z_v = _allsum(dot) + b_v
                    gate = 1.0 / (1.0 + jnp.exp(-z_v))

                    for c in range(DC):
                        ds_s = pl.ds(off + c * L, L)
                        acc[ds_s] = acc[ds_s] + x[c] * gate
                    for c in range(DC):
                        ds_m = pl.ds(off + D + c * L, L)
                        acc[ds_m] = jnp.maximum(acc[ds_m], x[c])
                return 0

            lax.cond(uniform, fast, slow, 0)
            return carry

        lax.fori_loop(0, GROUPS, group, 0)

    @pl.when(n_chunks > 0)
    def _prologue():
        start(0, fbufA, sbufA, semA)

    def pair(kk, carry):
        k0 = 2 * kk

        @pl.when(k0 + 1 < n_chunks)
        def _s1():
            start(k0 + 1, fbufB, sbufB, semB)
        wait(fbufA, sbufA, semA)
        process(k0, fbufA, sbufA)

        @pl.when(k0 + 2 < n_chunks)
        def _s2():
            start(k0 + 2, fbufA, sbufA, semA)

        @pl.when(k0 + 1 < n_chunks)
        def _p1():
            wait(fbufB, sbufB, semB)
            process(k0 + 1, fbufB, sbufB)
        return carry

    lax.fori_loop(0, (n_chunks + 1) // 2, pair, 0)

    pltpu.sync_copy(acc.at[pl.ds(0, SEG_PER_W * ROW_W)],
                    out_hbm.at[pl.ds(wid * SEG_PER_W * ROW_W,
                                     SEG_PER_W * ROW_W)])


@jax.jit
def kernel(feats, segment_ids, W, b):
    params = jnp.concatenate(
        [W.reshape(D), b.astype(jnp.float32),
         jnp.zeros((2 * L - 1,), jnp.float32)])
    seg_bounds = jnp.searchsorted(
        segment_ids,
        jnp.arange(0, NUM_SEGMENTS + 1, SEG_PER_W, dtype=jnp.int32),
    ).astype(jnp.int32)
    bounds = jnp.concatenate(
        [seg_bounds, jnp.full((15,), N, jnp.int32)])

    mesh = plsc.VectorSubcoreMesh(core_axis_name="c", subcore_axis_name="s",
                                  num_cores=2, num_subcores=16)
    run = pl.kernel(
        _body,
        out_type=jax.ShapeDtypeStruct((NUM_SEGMENTS * ROW_W,), jnp.float32),
        mesh=mesh,
        scratch_types=[
            pltpu.VMEM((CHUNK, D), jnp.float32),
            pltpu.VMEM((CHUNK, D), jnp.float32),
            pltpu.VMEM((CHUNK,), jnp.int32),
            pltpu.VMEM((CHUNK,), jnp.int32),
            pltpu.VMEM(((SEG_PER_W + 1) * ROW_W,), jnp.float32),
            pltpu.VMEM((D + 2 * L,), jnp.float32),
            pltpu.VMEM((3 * L,), jnp.int32),
            pltpu.SemaphoreType.DMA,
            pltpu.SemaphoreType.DMA,
        ],
        compiler_params=pltpu.CompilerParams(needs_layout_passes=False),
    )
    return run(feats, segment_ids, params, bounds).reshape(
        NUM_SEGMENTS, ROW_W)


# speedup vs baseline: 1.5082x; 1.0107x over previous
import jax
import jax.numpy as jnp
from jax import lax
from jax.experimental import pallas as pl
from jax.experimental.pallas import tpu as pltpu
from jax.experimental.pallas import tpu_sc as plsc

N = 100000
D = 128
NUM_SEGMENTS = 1024
NW = 32
SEG_PER_W = NUM_SEGMENTS // NW
CHUNK = 384
GROUPS = CHUNK // 16
L = 16
DC = D // L
ROW_W = 2 * D

_GDN = lax.GatherDimensionNumbers(
    offset_dims=(), collapsed_slice_dims=(0,), start_index_map=(0,))


def _perm(v, p):
    return lax.gather(v, p.reshape(L, 1), _GDN, (1,),
                      mode=lax.GatherScatterMode.PROMISE_IN_BOUNDS)


def _bcast_lane(v, j):
    return _perm(v, jnp.full((L,), j, dtype=jnp.int32))


def _allsum(v):
    lanes = jnp.arange(L, dtype=jnp.int32)
    for s in (8, 4, 2, 1):
        v = v + _perm(v, jnp.bitwise_xor(lanes, s))
    return v


def _body(feats_hbm, seg_hbm, params_hbm, bounds_hbm, out_hbm,
          fbufA, fbufB, sbufA, sbufB, acc, wbuf, bbuf, semA, semB):
    wid = lax.axis_index("s") * 2 + lax.axis_index("c")

    pltpu.sync_copy(params_hbm, wbuf)
    pltpu.sync_copy(bounds_hbm, bbuf)

    bv = bbuf[pl.ds(wid, L)]
    row_start = bv[0]
    row_end = bv[1]

    wv = [wbuf[pl.ds(c * L, L)] for c in range(DC)]
    b_v = _bcast_lane(wbuf[pl.ds(D, L)], 0)

    segbase = wid * SEG_PER_W
    zeros_v = jnp.zeros((L,), jnp.float32)
    neginf_v = jnp.full((L,), -jnp.inf, jnp.float32)

    def init_row(i, carry):
        for c in range(DC):
            acc[pl.ds(i * ROW_W + c * L, L)] = zeros_v
            acc[pl.ds(i * ROW_W + D + c * L, L)] = neginf_v
        return carry
    lax.fori_loop(0, SEG_PER_W + 1, init_row, 0)

    base0 = pl.multiple_of(jnp.bitwise_and(row_start, -8), 8)
    n_chunks = jnp.maximum(
        (row_end - base0 + CHUNK - 1) // CHUNK, 0)

    def chunk_base(k):
        return pl.multiple_of(jnp.minimum(base0 + k * CHUNK, N - CHUNK), 8)

    def start(k, fb, sb, sem):
        b = chunk_base(k)
        pltpu.async_copy(feats_hbm.at[pl.ds(b, CHUNK)], fb, sem)
        pltpu.async_copy(seg_hbm.at[pl.ds(b, CHUNK)], sb, sem)

    def wait(fb, sb, sem):
        pltpu.make_async_copy(feats_hbm.at[pl.ds(0, CHUNK)], fb, sem).wait()
        pltpu.make_async_copy(seg_hbm.at[pl.ds(0, CHUNK)], sb, sem).wait()

    def process(k, fb, sb):
        bk = base0 + k * CHUNK
        base = chunk_base(k)
        lo = jnp.maximum(bk, row_start)
        hi = jnp.minimum(bk + CHUNK, row_end)

        def group(g, carry):
            seg_vec = sb[pl.ds(g * L, L)]
            g_lo = base + g * L
            uniform = jnp.logical_and(
                jnp.all(seg_vec == _bcast_lane(seg_vec, 0)),
                jnp.logical_and(g_lo >= lo, g_lo + L <= hi))

            def fast(_):
                s_off = jnp.clip(seg_vec[0] - segbase, 0, SEG_PER_W - 1) \
                    * ROW_W
                gsum = [zeros_v] * DC
                gmax = [neginf_v] * DC
                for j in range(L):
                    row = g * L + j
                    x = [fb[row, pl.ds(c * L, L)] for c in range(DC)]
                    dot = x[0] * wv[0]
                    for c in range(1, DC):
                        dot = dot + x[c] * wv[c]
                    z_v = _allsum(dot) + b_v
                    gate = 1.0 / (1.0 + jnp.exp(-z_v))
                    for c in range(DC):
                        gsum[c] = gsum[c] + x[c] * gate
                        gmax[c] = jnp.maximum(gmax[c], x[c])
                for c in range(DC):
                    ds_s = pl.ds(s_off + c * L, L)
                    acc[ds_s] = acc[ds_s] + gsum[c]
                    ds_m = pl.ds(s_off + D + c * L, L)
                    acc[ds_m] = jnp.maximum(acc[ds_m], gmax[c])
                return 0

            def slow(_):
                for j in range(L):
                    rg = g_lo + j
                    m = jnp.logical_and(rg >= lo, rg < hi)
                    s_cl = jnp.clip(seg_vec[j] - segbase, 0, SEG_PER_W - 1)
                    off = jnp.where(m, s_cl, SEG_PER_W) * ROW_W

                    row = g * L + j
                    x = [fb[row, pl.ds(c * L, L)] for c in range(DC)]
                    dot = x[0] * wv[0]
                    for c in range(1, DC):
                        dot = dot + x[c] * wv[c]
                    z_v = _allsum(dot) + b_v
                    gate = 1.0 / (1.0 + jnp.exp(-z_v))

                    for c in range(DC):
                        ds_s = pl.ds(off + c * L, L)
                        acc[ds_s] = acc[ds_s] + x[c] * gate
                    for c in range(DC):
                        ds_m = pl.ds(off + D + c * L, L)
                        acc[ds_m] = jnp.maximum(acc[ds_m], x[c])
                return 0

            lax.cond(uniform, fast, slow, 0)
            return carry

        lax.fori_loop(0, GROUPS, group, 0)

    @pl.when(n_chunks > 0)
    def _prologue():
        start(0, fbufA, sbufA, semA)

    def pair(kk, carry):
        k0 = 2 * kk

        @pl.when(k0 + 1 < n_chunks)
        def _s1():
            start(k0 + 1, fbufB, sbufB, semB)
        wait(fbufA, sbufA, semA)
        process(k0, fbufA, sbufA)

        @pl.when(k0 + 2 < n_chunks)
        def _s2():
            start(k0 + 2, fbufA, sbufA, semA)

        @pl.when(k0 + 1 < n_chunks)
        def _p1():
            wait(fbufB, sbufB, semB)
            process(k0 + 1, fbufB, sbufB)
        return carry

    lax.fori_loop(0, (n_chunks + 1) // 2, pair, 0)

    pltpu.sync_copy(acc.at[pl.ds(0, SEG_PER_W * ROW_W)],
                    out_hbm.at[pl.ds(wid * SEG_PER_W * ROW_W,
                                     SEG_PER_W * ROW_W)])


@jax.jit
def kernel(feats, segment_ids, W, b):
    params = jnp.concatenate(
        [W.reshape(D), b.astype(jnp.float32),
         jnp.zeros((2 * L - 1,), jnp.float32)])
    seg_bounds = jnp.searchsorted(
        segment_ids,
        jnp.arange(0, NUM_SEGMENTS + 1, SEG_PER_W, dtype=jnp.int32),
    ).astype(jnp.int32)
    bounds = jnp.concatenate(
        [seg_bounds, jnp.full((15,), N, jnp.int32)])

    mesh = plsc.VectorSubcoreMesh(core_axis_name="c", subcore_axis_name="s",
                                  num_cores=2, num_subcores=16)
    run = pl.kernel(
        _body,
        out_type=jax.ShapeDtypeStruct((NUM_SEGMENTS * ROW_W,), jnp.float32),
        mesh=mesh,
        scratch_types=[
            pltpu.VMEM((CHUNK, D), jnp.float32),
            pltpu.VMEM((CHUNK, D), jnp.float32),
            pltpu.VMEM((CHUNK,), jnp.int32),
            pltpu.VMEM((CHUNK,), jnp.int32),
            pltpu.VMEM(((SEG_PER_W + 1) * ROW_W,), jnp.float32),
            pltpu.VMEM((D + 2 * L,), jnp.float32),
            pltpu.VMEM((3 * L,), jnp.int32),
            pltpu.SemaphoreType.DMA,
            pltpu.SemaphoreType.DMA,
        ],
        compiler_params=pltpu.CompilerParams(needs_layout_passes=False),
    )
    return run(feats, segment_ids, params, bounds).reshape(
        NUM_SEGMENTS, ROW_W)


# speedup vs baseline: 1.7354x; 1.1506x over previous
import jax
import jax.numpy as jnp
from jax import lax
from jax.experimental import pallas as pl
from jax.experimental.pallas import tpu as pltpu
from jax.experimental.pallas import tpu_sc as plsc

N = 100000
D = 128
NUM_SEGMENTS = 1024
NW = 32
SEG_PER_W = NUM_SEGMENTS // NW
CHUNK = 384
GROUPS = CHUNK // 16
L = 16
DC = D // L
ROW_W = 2 * D
TRASH_OFF = SEG_PER_W * ROW_W

_GDN = lax.GatherDimensionNumbers(
    offset_dims=(), collapsed_slice_dims=(0,), start_index_map=(0,))


def _perm(v, p):
    return lax.gather(v, p.reshape(L, 1), _GDN, (1,),
                      mode=lax.GatherScatterMode.PROMISE_IN_BOUNDS)


def _bcast_lane(v, j):
    return _perm(v, jnp.full((L,), j, dtype=jnp.int32))


def _allsum(v):
    lanes = jnp.arange(L, dtype=jnp.int32)
    for s in (8, 4, 2, 1):
        v = v + _perm(v, jnp.bitwise_xor(lanes, s))
    return v


def _body(feats_hbm, seg_hbm, params_hbm, bounds_hbm, out_hbm,
          fbufA, fbufB, sbufA, sbufB, acc, wbuf, bbuf, semA, semB):
    wid = lax.axis_index("s") * 2 + lax.axis_index("c")

    pltpu.sync_copy(params_hbm, wbuf)
    pltpu.sync_copy(bounds_hbm, bbuf)

    bv = bbuf[pl.ds(wid, L)]
    row_start = bv[0]
    row_end = bv[1]

    wv = [wbuf[pl.ds(c * L, L)] for c in range(DC)]
    b_v = _bcast_lane(wbuf[pl.ds(D, L)], 0)

    segbase = wid * SEG_PER_W
    zeros_v = jnp.zeros((L,), jnp.float32)
    neginf_v = jnp.full((L,), -jnp.inf, jnp.float32)

    def init_row(i, carry):
        for c in range(DC):
            acc[pl.ds(i * ROW_W + c * L, L)] = zeros_v
            acc[pl.ds(i * ROW_W + D + c * L, L)] = neginf_v
        return carry
    lax.fori_loop(0, SEG_PER_W + 1, init_row, 0)

    base0 = pl.multiple_of(jnp.bitwise_and(row_start, -8), 8)
    n_chunks = jnp.maximum(
        (row_end - base0 + CHUNK - 1) // CHUNK, 0)

    def chunk_base(k):
        return pl.multiple_of(jnp.minimum(base0 + k * CHUNK, N - CHUNK), 8)

    def start(k, fb, sb, sem):
        b = chunk_base(k)
        pltpu.async_copy(feats_hbm.at[pl.ds(b, CHUNK)], fb, sem)
        pltpu.async_copy(seg_hbm.at[pl.ds(b, CHUNK)], sb, sem)

    def wait(fb, sb, sem):
        pltpu.make_async_copy(feats_hbm.at[pl.ds(0, CHUNK)], fb, sem).wait()
        pltpu.make_async_copy(seg_hbm.at[pl.ds(0, CHUNK)], sb, sem).wait()


    def flush(cur_off, rs, rm):
        for c in range(DC):
            ds_s = pl.ds(cur_off + c * L, L)
            acc[ds_s] = acc[ds_s] + rs[c]
            ds_m = pl.ds(cur_off + D + c * L, L)
            acc[ds_m] = jnp.maximum(acc[ds_m], rm[c])

    def identity_regs():
        return [zeros_v] * DC, [neginf_v] * DC

    def process(k, fb, sb, carry):
        bk = base0 + k * CHUNK
        base = chunk_base(k)
        lo = jnp.maximum(bk, row_start)
        hi = jnp.minimum(bk + CHUNK, row_end)

        def group(g, carry):
            cur_off, rs, rm = carry[0], list(carry[1]), list(carry[2])
            seg_vec = sb[pl.ds(g * L, L)]
            g_lo = base + g * L
            uniform = jnp.logical_and(
                jnp.all(seg_vec == _bcast_lane(seg_vec, 0)),
                jnp.logical_and(g_lo >= lo, g_lo + L <= hi))

            def fast(carry):
                cur_off, rs, rm = carry[0], list(carry[1]), list(carry[2])
                off_new = jnp.clip(seg_vec[0] - segbase, 0, SEG_PER_W - 1) \
                    * ROW_W

                def keep(c):
                    return (c[0], tuple(c[1]), tuple(c[2]))

                def switch(c):
                    flush(c[0], c[1], c[2])
                    zs, zm = identity_regs()
                    return (off_new, tuple(zs), tuple(zm))

                cur_off, rs, rm = lax.cond(
                    off_new == cur_off, keep, switch,
                    (cur_off, tuple(rs), tuple(rm)))
                rs, rm = list(rs), list(rm)
                for j in range(L):
                    row = g * L + j
                    x = [fb[row, pl.ds(c * L, L)] for c in range(DC)]
                    dot = x[0] * wv[0]
                    for c in range(1, DC):
                        dot = dot + x[c] * wv[c]
                    z_v = _allsum(dot) + b_v
                    gate = 1.0 / (1.0 + jnp.exp(-z_v))
                    for c in range(DC):
                        rs[c] = rs[c] + x[c] * gate
                        rm[c] = jnp.maximum(rm[c], x[c])
                return (off_new, tuple(rs), tuple(rm))

            def slow(carry):
                flush(carry[0], carry[1], carry[2])
                for j in range(L):
                    rg = g_lo + j
                    m = jnp.logical_and(rg >= lo, rg < hi)
                    s_cl = jnp.clip(seg_vec[j] - segbase, 0, SEG_PER_W - 1)
                    off = jnp.where(m, s_cl, SEG_PER_W) * ROW_W

                    row = g * L + j
                    x = [fb[row, pl.ds(c * L, L)] for c in range(DC)]
                    dot = x[0] * wv[0]
                    for c in range(1, DC):
                        dot = dot + x[c] * wv[c]
                    z_v = _allsum(dot) + b_v
                    gate = 1.0 / (1.0 + jnp.exp(-z_v))

                    for c in range(DC):
                        ds_s = pl.ds(off + c * L, L)
                        acc[ds_s] = acc[ds_s] + x[c] * gate
                    for c in range(DC):
                        ds_m = pl.ds(off + D + c * L, L)
                        acc[ds_m] = jnp.maximum(acc[ds_m], x[c])
                zs, zm = identity_regs()
                return (TRASH_OFF, tuple(zs), tuple(zm))

            return lax.cond(uniform, fast, slow,
                            (cur_off, tuple(rs), tuple(rm)))

        return lax.fori_loop(0, GROUPS, group, carry)

    @pl.when(n_chunks > 0)
    def _prologue():
        start(0, fbufA, sbufA, semA)

    def pair(kk, carry):
        k0 = 2 * kk

        @pl.when(k0 + 1 < n_chunks)
        def _s1():
            start(k0 + 1, fbufB, sbufB, semB)
        wait(fbufA, sbufA, semA)
        carry = process(k0, fbufA, sbufA, carry)

        @pl.when(k0 + 2 < n_chunks)
        def _s2():
            start(k0 + 2, fbufA, sbufA, semA)

        def do_b(c):
            wait(fbufB, sbufB, semB)
            return process(k0 + 1, fbufB, sbufB, c)

        return lax.cond(k0 + 1 < n_chunks, do_b, lambda c: c, carry)

    zs0, zm0 = identity_regs()
    carry0 = (jnp.int32(TRASH_OFF), tuple(zs0), tuple(zm0))
    final = lax.fori_loop(0, (n_chunks + 1) // 2, pair, carry0)
    flush(final[0], final[1], final[2])

    pltpu.sync_copy(acc.at[pl.ds(0, SEG_PER_W * ROW_W)],
                    out_hbm.at[pl.ds(wid * SEG_PER_W * ROW_W,
                                     SEG_PER_W * ROW_W)])


@jax.jit
def kernel(feats, segment_ids, W, b):
    params = jnp.concatenate(
        [W.reshape(D), b.astype(jnp.float32),
         jnp.zeros((2 * L - 1,), jnp.float32)])
    seg_bounds = jnp.searchsorted(
        segment_ids,
        jnp.arange(0, NUM_SEGMENTS + 1, SEG_PER_W, dtype=jnp.int32),
    ).astype(jnp.int32)
    bounds = jnp.concatenate(
        [seg_bounds, jnp.full((15,), N, jnp.int32)])

    mesh = plsc.VectorSubcoreMesh(core_axis_name="c", subcore_axis_name="s",
                                  num_cores=2, num_subcores=16)
    run = pl.kernel(
        _body,
        out_type=jax.ShapeDtypeStruct((NUM_SEGMENTS * ROW_W,), jnp.float32),
        mesh=mesh,
        scratch_types=[
            pltpu.VMEM((CHUNK, D), jnp.float32),
            pltpu.VMEM((CHUNK, D), jnp.float32),
            pltpu.VMEM((CHUNK,), jnp.int32),
            pltpu.VMEM((CHUNK,), jnp.int32),
            pltpu.VMEM(((SEG_PER_W + 1) * ROW_W,), jnp.float32),
            pltpu.VMEM((D + 2 * L,), jnp.float32),
            pltpu.VMEM((3 * L,), jnp.int32),
            pltpu.SemaphoreType.DMA,
            pltpu.SemaphoreType.DMA,
        ],
        compiler_params=pltpu.CompilerParams(needs_layout_passes=False),
    )
    return run(feats, segment_ids, params, bounds).reshape(
        NUM_SEGMENTS, ROW_W)
